# initial kernel scaffold (unmeasured)
import jax
import jax.numpy as jnp
from jax import lax
from jax.experimental import pallas as pl
from jax.experimental.pallas import tpu as pltpu

S, D, H, Dh, Dr = 1024, 2048, 16, 128, 32
DC_SH = 128
SCALE = (Dh + Dr) ** -0.5


def _kv_exchange_body(x_ref, wdkv_ref, wuk_ref, wuv_ref, k_ref, v_ref,
                      c_ref, pc_ref, pwuk_ref, pwuv_ref,
                      send_sems, recv_sems):
    my_x = lax.axis_index("x")
    my_y = lax.axis_index("y")
    my_z = lax.axis_index("z")
    peer = (my_x, my_y, 1 - my_z)

    barrier = pltpu.get_barrier_semaphore()
    pl.semaphore_signal(barrier, inc=1, device_id=peer,
                        device_id_type=pl.DeviceIdType.MESH)
    pl.semaphore_wait(barrier, 1)

    rdma_wuk = pltpu.make_async_remote_copy(
        src_ref=wuk_ref, dst_ref=pwuk_ref,
        send_sem=send_sems.at[0], recv_sem=recv_sems.at[0],
        device_id=peer, device_id_type=pl.DeviceIdType.MESH)
    rdma_wuk.start()
    rdma_wuv = pltpu.make_async_remote_copy(
        src_ref=wuv_ref, dst_ref=pwuv_ref,
        send_sem=send_sems.at[1], recv_sem=recv_sems.at[1],
        device_id=peer, device_id_type=pl.DeviceIdType.MESH)
    rdma_wuv.start()

    c_ref[...] = jnp.dot(x_ref[...], wdkv_ref[...],
                         preferred_element_type=jnp.float32)

    rdma_c = pltpu.make_async_remote_copy(
        src_ref=c_ref, dst_ref=pc_ref,
        send_sem=send_sems.at[2], recv_sem=recv_sems.at[2],
        device_id=peer, device_id_type=pl.DeviceIdType.MESH)
    rdma_c.start()

    k_ref[...] = jnp.dot(c_ref[...], wuk_ref[...],
                         preferred_element_type=jnp.float32)
    v_ref[...] = jnp.dot(c_ref[...], wuv_ref[...],
                         preferred_element_type=jnp.float32)

    rdma_wuk.wait()
    rdma_wuv.wait()
    rdma_c.wait()

    k_ref[...] += jnp.dot(pc_ref[...], pwuk_ref[...],
                          preferred_element_type=jnp.float32)
    v_ref[...] += jnp.dot(pc_ref[...], pwuv_ref[...],
                          preferred_element_type=jnp.float32)


def _kv_exchange(x2d, Wdkv, Wuk, Wuv):
    return pl.pallas_call(
        _kv_exchange_body,
        out_shape=(jax.ShapeDtypeStruct((S, D), jnp.float32),
                   jax.ShapeDtypeStruct((S, D), jnp.float32)),
        in_specs=[pl.BlockSpec(memory_space=pltpu.VMEM)] * 4,
        out_specs=(pl.BlockSpec(memory_space=pltpu.VMEM),
                   pl.BlockSpec(memory_space=pltpu.VMEM)),
        scratch_shapes=[
            pltpu.VMEM((S, DC_SH), jnp.float32),
            pltpu.VMEM((S, DC_SH), jnp.float32),
            pltpu.VMEM((DC_SH, D), jnp.float32),
            pltpu.VMEM((DC_SH, D), jnp.float32),
            pltpu.SemaphoreType.DMA((3,)),
            pltpu.SemaphoreType.DMA((3,)),
        ],
        compiler_params=pltpu.CompilerParams(collective_id=0),
    )(x2d, Wdkv, Wuk, Wuv)


def _qproj_body(x_ref, wq_ref, wqr_ref, wkr_ref, q_ref, qr_ref, kr_ref):
    q_ref[...] = jnp.dot(x_ref[...], wq_ref[...],
                         preferred_element_type=jnp.float32)
    qr_ref[...] = jnp.dot(x_ref[...], wqr_ref[...],
                          preferred_element_type=jnp.float32)
    kr_ref[...] = jnp.dot(x_ref[...], wkr_ref[...],
                          preferred_element_type=jnp.float32)


def _qproj(x2d, Wq, Wqr, Wkr):
    return pl.pallas_call(
        _qproj_body,
        out_shape=(jax.ShapeDtypeStruct((S, D), jnp.float32),
                   jax.ShapeDtypeStruct((S, H * Dr), jnp.float32),
                   jax.ShapeDtypeStruct((S, Dr), jnp.float32)),
        in_specs=[pl.BlockSpec(memory_space=pltpu.VMEM)] * 4,
        out_specs=(pl.BlockSpec(memory_space=pltpu.VMEM),) * 3,
    )(x2d, Wq, Wqr, Wkr)


def _attn_body(q_ref, k_ref, v_ref, qr_ref, kr_ref, o_ref):
    s = lax.dot_general(q_ref[...], k_ref[...],
                        (((1,), (1,)), ((), ())),
                        preferred_element_type=jnp.float32)
    s += lax.dot_general(qr_ref[...], kr_ref[...],
                         (((1,), (1,)), ((), ())),
                         preferred_element_type=jnp.float32)
    s *= SCALE
    m = jnp.max(s, axis=1, keepdims=True)
    p = jnp.exp(s - m)
    p = p / jnp.sum(p, axis=1, keepdims=True)
    o_ref[...] = jnp.dot(p, v_ref[...], preferred_element_type=jnp.float32)


def _attn(q, k, v, qr, kr):
    return pl.pallas_call(
        _attn_body,
        grid=(H,),
        out_shape=jax.ShapeDtypeStruct((S, D), jnp.float32),
        in_specs=[
            pl.BlockSpec((S, Dh), lambda h: (0, h)),
            pl.BlockSpec((S, Dh), lambda h: (0, h)),
            pl.BlockSpec((S, Dh), lambda h: (0, h)),
            pl.BlockSpec((S, Dr), lambda h: (0, h)),
            pl.BlockSpec((S, Dr), lambda h: (0, 0)),
        ],
        out_specs=pl.BlockSpec((S, Dh), lambda h: (0, h)),
        compiler_params=pltpu.CompilerParams(
            dimension_semantics=("arbitrary",)),
    )(q, k, v, qr, kr)


def _oproj_body(o_ref, wo_ref, out_ref):
    out_ref[...] = jnp.dot(o_ref[...], wo_ref[...],
                           preferred_element_type=jnp.float32)


def _oproj(o, Wo):
    return pl.pallas_call(
        _oproj_body,
        out_shape=jax.ShapeDtypeStruct((S, D), jnp.float32),
        in_specs=[pl.BlockSpec(memory_space=pltpu.VMEM)] * 2,
        out_specs=pl.BlockSpec(memory_space=pltpu.VMEM),
    )(o, Wo)


def kernel(x, Wdkv, Wuk, Wuv, Wq, Wqr, Wkr, Wo):
    x2d = x.reshape(S, D)
    k, v = _kv_exchange(x2d, Wdkv, Wuk, Wuv)
    q, qr, kr = _qproj(x2d, Wq, Wqr, Wkr)
    o = _attn(q, k, v, qr, kr)
    out = _oproj(o, Wo)
    return out.reshape(1, S, D)


# baseline (device time: 153301 ns/iter reference)
import jax
import jax.numpy as jnp
from jax import lax
from jax.experimental import pallas as pl
from jax.experimental.pallas import tpu as pltpu

S, D, H, Dh, Dr = 1024, 2048, 16, 128, 32
DC_SH = 128
SCALE = (Dh + Dr) ** -0.5


def _kv_exchange_body(x_ref, wdkv_ref, wuk_ref, wuv_ref, k_ref, v_ref,
                      c_ref, pc_ref, pwuk_ref, pwuv_ref,
                      send_sems, recv_sems):
    my_x = lax.axis_index("x")
    my_y = lax.axis_index("y")
    my_z = lax.axis_index("z")
    peer = (my_x, my_y, 1 - my_z)

    barrier = pltpu.get_barrier_semaphore()
    pl.semaphore_signal(barrier, inc=1, device_id=peer,
                        device_id_type=pl.DeviceIdType.MESH)
    pl.semaphore_wait(barrier, 1)

    rdma_wuk = pltpu.make_async_remote_copy(
        src_ref=wuk_ref, dst_ref=pwuk_ref,
        send_sem=send_sems.at[0], recv_sem=recv_sems.at[0],
        device_id=peer, device_id_type=pl.DeviceIdType.MESH)
    rdma_wuk.start()
    rdma_wuv = pltpu.make_async_remote_copy(
        src_ref=wuv_ref, dst_ref=pwuv_ref,
        send_sem=send_sems.at[1], recv_sem=recv_sems.at[1],
        device_id=peer, device_id_type=pl.DeviceIdType.MESH)
    rdma_wuv.start()

    c_ref[...] = jnp.dot(x_ref[...], wdkv_ref[...],
                         preferred_element_type=jnp.float32)

    rdma_c = pltpu.make_async_remote_copy(
        src_ref=c_ref, dst_ref=pc_ref,
        send_sem=send_sems.at[2], recv_sem=recv_sems.at[2],
        device_id=peer, device_id_type=pl.DeviceIdType.MESH)
    rdma_c.start()

    k_ref[...] = jnp.dot(c_ref[...], wuk_ref[...],
                         preferred_element_type=jnp.float32)
    v_ref[...] = jnp.dot(c_ref[...], wuv_ref[...],
                         preferred_element_type=jnp.float32)

    rdma_wuk.wait()
    rdma_wuv.wait()
    rdma_c.wait()

    k_ref[...] += jnp.dot(pc_ref[...], pwuk_ref[...],
                          preferred_element_type=jnp.float32)
    v_ref[...] += jnp.dot(pc_ref[...], pwuv_ref[...],
                          preferred_element_type=jnp.float32)


def _kv_exchange(x2d, Wdkv, Wuk, Wuv):
    return pl.pallas_call(
        _kv_exchange_body,
        out_shape=(jax.ShapeDtypeStruct((S, D), jnp.float32),
                   jax.ShapeDtypeStruct((S, D), jnp.float32)),
        in_specs=[pl.BlockSpec(memory_space=pltpu.VMEM)] * 4,
        out_specs=(pl.BlockSpec(memory_space=pltpu.VMEM),
                   pl.BlockSpec(memory_space=pltpu.VMEM)),
        scratch_shapes=[
            pltpu.VMEM((S, DC_SH), jnp.float32),
            pltpu.VMEM((S, DC_SH), jnp.float32),
            pltpu.VMEM((DC_SH, D), jnp.float32),
            pltpu.VMEM((DC_SH, D), jnp.float32),
            pltpu.SemaphoreType.DMA((3,)),
            pltpu.SemaphoreType.DMA((3,)),
        ],
        compiler_params=pltpu.CompilerParams(collective_id=0),
    )(x2d, Wdkv, Wuk, Wuv)


def _qproj_body(x_ref, wq_ref, wqr_ref, wkr_ref, q_ref, qr_ref, kr_ref):
    q_ref[...] = jnp.dot(x_ref[...], wq_ref[...],
                         preferred_element_type=jnp.float32)
    qr = jnp.dot(x_ref[...], wqr_ref[...],
                 preferred_element_type=jnp.float32)
    for h in range(H):
        qr_ref[h, :, :] = qr[:, h * Dr:(h + 1) * Dr]
    kr_ref[...] = jnp.dot(x_ref[...], wkr_ref[...],
                          preferred_element_type=jnp.float32)


def _qproj(x2d, Wq, Wqr, Wkr):
    return pl.pallas_call(
        _qproj_body,
        out_shape=(jax.ShapeDtypeStruct((S, D), jnp.float32),
                   jax.ShapeDtypeStruct((H, S, Dr), jnp.float32),
                   jax.ShapeDtypeStruct((S, Dr), jnp.float32)),
        in_specs=[pl.BlockSpec(memory_space=pltpu.VMEM)] * 4,
        out_specs=(pl.BlockSpec(memory_space=pltpu.VMEM),) * 3,
    )(x2d, Wq, Wqr, Wkr)


def _attn_body(q_ref, k_ref, v_ref, qr_ref, kr_ref, o_ref):
    s = lax.dot_general(q_ref[...], k_ref[...],
                        (((1,), (1,)), ((), ())),
                        preferred_element_type=jnp.float32)
    s += lax.dot_general(qr_ref[0], kr_ref[...],
                         (((1,), (1,)), ((), ())),
                         preferred_element_type=jnp.float32)
    s *= SCALE
    m = jnp.max(s, axis=1, keepdims=True)
    p = jnp.exp(s - m)
    p = p / jnp.sum(p, axis=1, keepdims=True)
    o_ref[...] = jnp.dot(p, v_ref[...], preferred_element_type=jnp.float32)


def _attn(q, k, v, qr, kr):
    return pl.pallas_call(
        _attn_body,
        grid=(H,),
        out_shape=jax.ShapeDtypeStruct((S, D), jnp.float32),
        in_specs=[
            pl.BlockSpec((S, Dh), lambda h: (0, h)),
            pl.BlockSpec((S, Dh), lambda h: (0, h)),
            pl.BlockSpec((S, Dh), lambda h: (0, h)),
            pl.BlockSpec((1, S, Dr), lambda h: (h, 0, 0)),
            pl.BlockSpec((S, Dr), lambda h: (0, 0)),
        ],
        out_specs=pl.BlockSpec((S, Dh), lambda h: (0, h)),
        compiler_params=pltpu.CompilerParams(
            dimension_semantics=("arbitrary",)),
    )(q, k, v, qr, kr)


def _oproj_body(o_ref, wo_ref, out_ref):
    out_ref[...] = jnp.dot(o_ref[...], wo_ref[...],
                           preferred_element_type=jnp.float32)


def _oproj(o, Wo):
    return pl.pallas_call(
        _oproj_body,
        out_shape=jax.ShapeDtypeStruct((S, D), jnp.float32),
        in_specs=[pl.BlockSpec(memory_space=pltpu.VMEM)] * 2,
        out_specs=pl.BlockSpec(memory_space=pltpu.VMEM),
    )(o, Wo)


def kernel(x, Wdkv, Wuk, Wuv, Wq, Wqr, Wkr, Wo):
    x2d = x.reshape(S, D)
    k, v = _kv_exchange(x2d, Wdkv, Wuk, Wuv)
    q, qr, kr = _qproj(x2d, Wq, Wqr, Wkr)
    o = _attn(q, k, v, qr, kr)
    out = _oproj(o, Wo)
    return out.reshape(1, S, D)


# device time: 124353 ns/iter; 1.2328x vs baseline; 1.2328x over previous
import jax
import jax.numpy as jnp
from jax import lax
from jax.experimental import pallas as pl
from jax.experimental.pallas import tpu as pltpu

S, D, H, Dh, Dr = 1024, 2048, 16, 128, 32
DC_SH = 128
SCALE = (Dh + Dr) ** -0.5


def _proj_exchange_body(x_ref, wdkv_ref, wuk_ref, wuv_ref,
                        wq_ref, wqrt_ref, wkr_ref,
                        k_ref, v_ref, q_ref, qrt_ref, kr_ref,
                        c_ref, pc_ref, pwuk_ref, pwuv_ref,
                        k_tmp, v_tmp, q_tmp, qrt_tmp, kr_tmp,
                        send_sems, recv_sems, out_sems):
    my_x = lax.axis_index("x")
    my_y = lax.axis_index("y")
    my_z = lax.axis_index("z")
    peer = (my_x, my_y, 1 - my_z)

    barrier = pltpu.get_barrier_semaphore()
    pl.semaphore_signal(barrier, inc=1, device_id=peer,
                        device_id_type=pl.DeviceIdType.MESH)
    pl.semaphore_wait(barrier, 1)

    rdma_wuk = pltpu.make_async_remote_copy(
        src_ref=wuk_ref, dst_ref=pwuk_ref,
        send_sem=send_sems.at[0], recv_sem=recv_sems.at[0],
        device_id=peer, device_id_type=pl.DeviceIdType.MESH)
    rdma_wuk.start()
    rdma_wuv = pltpu.make_async_remote_copy(
        src_ref=wuv_ref, dst_ref=pwuv_ref,
        send_sem=send_sems.at[1], recv_sem=recv_sems.at[1],
        device_id=peer, device_id_type=pl.DeviceIdType.MESH)
    rdma_wuv.start()

    c_ref[...] = jnp.dot(x_ref[...], wdkv_ref[...],
                         preferred_element_type=jnp.float32)

    rdma_c = pltpu.make_async_remote_copy(
        src_ref=c_ref, dst_ref=pc_ref,
        send_sem=send_sems.at[2], recv_sem=recv_sems.at[2],
        device_id=peer, device_id_type=pl.DeviceIdType.MESH)
    rdma_c.start()

    q_tmp[...] = jnp.dot(x_ref[...], wq_ref[...],
                         preferred_element_type=jnp.float32) * SCALE
    qrt_tmp[...] = lax.dot_general(
        wqrt_ref[...], x_ref[...], (((1,), (1,)), ((), ())),
        preferred_element_type=jnp.float32) * SCALE
    kr_tmp[...] = jnp.dot(x_ref[...], wkr_ref[...],
                          preferred_element_type=jnp.float32)
    cp_q = pltpu.make_async_copy(q_tmp, q_ref, out_sems.at[0])
    cp_q.start()
    cp_qrt = pltpu.make_async_copy(qrt_tmp, qrt_ref, out_sems.at[1])
    cp_qrt.start()
    cp_kr = pltpu.make_async_copy(kr_tmp, kr_ref, out_sems.at[2])
    cp_kr.start()

    k_tmp[...] = jnp.dot(c_ref[...], wuk_ref[...],
                         preferred_element_type=jnp.float32)
    v_tmp[...] = jnp.dot(c_ref[...], wuv_ref[...],
                         preferred_element_type=jnp.float32)

    rdma_wuk.wait()
    rdma_c.wait()
    k_tmp[...] += jnp.dot(pc_ref[...], pwuk_ref[...],
                          preferred_element_type=jnp.float32)
    cp_k = pltpu.make_async_copy(k_tmp, k_ref, out_sems.at[3])
    cp_k.start()

    rdma_wuv.wait()
    v_tmp[...] += jnp.dot(pc_ref[...], pwuv_ref[...],
                          preferred_element_type=jnp.float32)
    cp_v = pltpu.make_async_copy(v_tmp, v_ref, out_sems.at[4])
    cp_v.start()

    cp_q.wait()
    cp_qrt.wait()
    cp_kr.wait()
    cp_k.wait()
    cp_v.wait()


def _proj_exchange(x2d, Wdkv, Wuk, Wuv, Wq, WqrT, Wkr):
    return pl.pallas_call(
        _proj_exchange_body,
        out_shape=(jax.ShapeDtypeStruct((S, D), jnp.float32),
                   jax.ShapeDtypeStruct((S, D), jnp.float32),
                   jax.ShapeDtypeStruct((S, D), jnp.float32),
                   jax.ShapeDtypeStruct((H * Dr, S), jnp.float32),
                   jax.ShapeDtypeStruct((S, Dr), jnp.float32)),
        in_specs=[pl.BlockSpec(memory_space=pltpu.VMEM)] * 7,
        out_specs=(pl.BlockSpec(memory_space=pl.ANY),) * 5,
        scratch_shapes=[
            pltpu.VMEM((S, DC_SH), jnp.float32),
            pltpu.VMEM((S, DC_SH), jnp.float32),
            pltpu.VMEM((DC_SH, D), jnp.float32),
            pltpu.VMEM((DC_SH, D), jnp.float32),
            pltpu.VMEM((S, D), jnp.float32),
            pltpu.VMEM((S, D), jnp.float32),
            pltpu.VMEM((S, D), jnp.float32),
            pltpu.VMEM((H * Dr, S), jnp.float32),
            pltpu.VMEM((S, Dr), jnp.float32),
            pltpu.SemaphoreType.DMA((3,)),
            pltpu.SemaphoreType.DMA((3,)),
            pltpu.SemaphoreType.DMA((5,)),
        ],
        compiler_params=pltpu.CompilerParams(
            collective_id=0,
            vmem_limit_bytes=100 * 1024 * 1024,
        ),
    )(x2d, Wdkv, Wuk, Wuv, Wq, WqrT, Wkr)


def _attn_body(q_ref, k_ref, v_ref, qrt_ref, kr_ref, o_ref):
    s = lax.dot_general(q_ref[...], k_ref[...],
                        (((1,), (1,)), ((), ())),
                        preferred_element_type=jnp.float32)
    s += lax.dot_general(qrt_ref[...], kr_ref[...],
                         (((0,), (1,)), ((), ())),
                         preferred_element_type=jnp.float32)
    p = jnp.exp(s)
    denom = jnp.sum(p, axis=1, keepdims=True)
    o = jnp.dot(p, v_ref[...], preferred_element_type=jnp.float32)
    o_ref[...] = o / denom


def _attn(q, k, v, qrt, kr):
    return pl.pallas_call(
        _attn_body,
        grid=(H,),
        out_shape=jax.ShapeDtypeStruct((S, D), jnp.float32),
        in_specs=[
            pl.BlockSpec((S, Dh), lambda h: (0, h)),
            pl.BlockSpec((S, Dh), lambda h: (0, h)),
            pl.BlockSpec((S, Dh), lambda h: (0, h)),
            pl.BlockSpec((Dr, S), lambda h: (h, 0)),
            pl.BlockSpec((S, Dr), lambda h: (0, 0)),
        ],
        out_specs=pl.BlockSpec((S, Dh), lambda h: (0, h)),
        compiler_params=pltpu.CompilerParams(
            dimension_semantics=("arbitrary",)),
    )(q, k, v, qrt, kr)


def _oproj_body(o_ref, wo_ref, out_ref):
    out_ref[...] = jnp.dot(o_ref[...], wo_ref[...],
                           preferred_element_type=jnp.float32)


def _oproj(o, Wo):
    return pl.pallas_call(
        _oproj_body,
        out_shape=jax.ShapeDtypeStruct((S, D), jnp.float32),
        in_specs=[pl.BlockSpec(memory_space=pltpu.VMEM)] * 2,
        out_specs=pl.BlockSpec(memory_space=pltpu.VMEM),
    )(o, Wo)


def kernel(x, Wdkv, Wuk, Wuv, Wq, Wqr, Wkr, Wo):
    x2d = x.reshape(S, D)
    k, v, q, qrt, kr = _proj_exchange(x2d, Wdkv, Wuk, Wuv, Wq,
                                      Wqr.T, Wkr)
    o = _attn(q, k, v, qrt, kr)
    out = _oproj(o, Wo)
    return out.reshape(1, S, D)


# device time: 122003 ns/iter; 1.2565x vs baseline; 1.0193x over previous
import jax
import jax.numpy as jnp
from jax import lax
from jax.experimental import pallas as pl
from jax.experimental.pallas import tpu as pltpu

S, D, H, Dh, Dr = 1024, 2048, 16, 128, 32
DC_SH = 128
SCALE = (Dh + Dr) ** -0.5


def _proj_exchange_body(x_ref, wdkv_ref, wuk_ref, wuv_ref,
                        wqr_ref, wkr_ref,
                        k_ref, v_ref, qrt_ref, kr_ref,
                        c_ref, c_bf, wuk_bf, wuv_bf, pc_bf, pwuk_bf,
                        pwuv_bf, qrt_tmp, kr_tmp,
                        send_sems, recv_sems, out_sems):
    my_x = lax.axis_index("x")
    my_y = lax.axis_index("y")
    my_z = lax.axis_index("z")
    peer = (my_x, my_y, 1 - my_z)
    x = x_ref[0]

    barrier = pltpu.get_barrier_semaphore()
    pl.semaphore_signal(barrier, inc=1, device_id=peer,
                        device_id_type=pl.DeviceIdType.MESH)
    pl.semaphore_wait(barrier, 1)

    wuk_bf[...] = wuk_ref[...].astype(jnp.bfloat16)
    rdma_wuk = pltpu.make_async_remote_copy(
        src_ref=wuk_bf, dst_ref=pwuk_bf,
        send_sem=send_sems.at[0], recv_sem=recv_sems.at[0],
        device_id=peer, device_id_type=pl.DeviceIdType.MESH)
    rdma_wuk.start()
    wuv_bf[...] = wuv_ref[...].astype(jnp.bfloat16)
    rdma_wuv = pltpu.make_async_remote_copy(
        src_ref=wuv_bf, dst_ref=pwuv_bf,
        send_sem=send_sems.at[1], recv_sem=recv_sems.at[1],
        device_id=peer, device_id_type=pl.DeviceIdType.MESH)
    rdma_wuv.start()

    c_ref[...] = jnp.dot(x, wdkv_ref[...],
                         preferred_element_type=jnp.float32)
    c_bf[...] = c_ref[...].astype(jnp.bfloat16)

    rdma_c = pltpu.make_async_remote_copy(
        src_ref=c_bf, dst_ref=pc_bf,
        send_sem=send_sems.at[2], recv_sem=recv_sems.at[2],
        device_id=peer, device_id_type=pl.DeviceIdType.MESH)
    rdma_c.start()

    qrt_tmp[...] = lax.dot_general(
        wqr_ref[...], x, (((0,), (1,)), ((), ())),
        preferred_element_type=jnp.float32) * SCALE
    kr_tmp[...] = jnp.dot(x, wkr_ref[...],
                          preferred_element_type=jnp.float32)
    cp_qrt = pltpu.make_async_copy(qrt_tmp, qrt_ref, out_sems.at[0])
    cp_qrt.start()
    cp_kr = pltpu.make_async_copy(kr_tmp, kr_ref, out_sems.at[1])
    cp_kr.start()

    k_ref[...] = jnp.dot(c_ref[...], wuk_ref[...],
                         preferred_element_type=jnp.float32)
    v_ref[...] = jnp.dot(c_ref[...], wuv_ref[...],
                         preferred_element_type=jnp.float32)

    rdma_wuk.wait()
    rdma_c.wait()
    k_ref[...] += jnp.dot(pc_bf[...], pwuk_bf[...],
                          preferred_element_type=jnp.float32)

    rdma_wuv.wait()
    v_ref[...] += jnp.dot(pc_bf[...], pwuv_bf[...],
                          preferred_element_type=jnp.float32)

    cp_qrt.wait()
    cp_kr.wait()


def _proj_exchange(x, Wdkv, Wuk, Wuv, Wqr, Wkr):
    return pl.pallas_call(
        _proj_exchange_body,
        out_shape=(jax.ShapeDtypeStruct((S, D), jnp.float32),
                   jax.ShapeDtypeStruct((S, D), jnp.float32),
                   jax.ShapeDtypeStruct((H * Dr, S), jnp.float32),
                   jax.ShapeDtypeStruct((S, Dr), jnp.float32)),
        in_specs=[pl.BlockSpec(memory_space=pltpu.VMEM)] * 6,
        out_specs=(pl.BlockSpec(memory_space=pltpu.VMEM),
                   pl.BlockSpec(memory_space=pltpu.VMEM),
                   pl.BlockSpec(memory_space=pl.ANY),
                   pl.BlockSpec(memory_space=pl.ANY)),
        scratch_shapes=[
            pltpu.VMEM((S, DC_SH), jnp.float32),
            pltpu.VMEM((S, DC_SH), jnp.bfloat16),
            pltpu.VMEM((DC_SH, D), jnp.bfloat16),
            pltpu.VMEM((DC_SH, D), jnp.bfloat16),
            pltpu.VMEM((S, DC_SH), jnp.bfloat16),
            pltpu.VMEM((DC_SH, D), jnp.bfloat16),
            pltpu.VMEM((DC_SH, D), jnp.bfloat16),
            pltpu.VMEM((H * Dr, S), jnp.float32),
            pltpu.VMEM((S, Dr), jnp.float32),
            pltpu.SemaphoreType.DMA((3,)),
            pltpu.SemaphoreType.DMA((3,)),
            pltpu.SemaphoreType.DMA((2,)),
        ],
        compiler_params=pltpu.CompilerParams(
            collective_id=0,
            vmem_limit_bytes=100 * 1024 * 1024,
        ),
    )(x, Wdkv, Wuk, Wuv, Wqr, Wkr)


def _qproj_body(x_ref, wq_ref, q_ref):
    q_ref[...] = jnp.dot(x_ref[0], wq_ref[...],
                         preferred_element_type=jnp.float32) * SCALE


def _qproj(x, Wq):
    return pl.pallas_call(
        _qproj_body,
        out_shape=jax.ShapeDtypeStruct((S, D), jnp.float32),
        in_specs=[pl.BlockSpec(memory_space=pltpu.VMEM)] * 2,
        out_specs=pl.BlockSpec(memory_space=pltpu.VMEM),
    )(x, Wq)


def _attn_body(q_ref, k_ref, v_ref, qrt_ref, kr_ref, o_ref):
    s = lax.dot_general(q_ref[...], k_ref[...],
                        (((1,), (1,)), ((), ())),
                        preferred_element_type=jnp.float32)
    s += lax.dot_general(qrt_ref[...], kr_ref[...],
                         (((0,), (1,)), ((), ())),
                         preferred_element_type=jnp.float32)
    p = jnp.exp(s)
    denom = jnp.sum(p, axis=1, keepdims=True)
    o = jnp.dot(p, v_ref[...], preferred_element_type=jnp.float32)
    o_ref[...] = o / denom


def _attn(q, k, v, qrt, kr):
    return pl.pallas_call(
        _attn_body,
        grid=(H,),
        out_shape=jax.ShapeDtypeStruct((S, D), jnp.float32),
        in_specs=[
            pl.BlockSpec((S, Dh), lambda h: (0, h)),
            pl.BlockSpec((S, Dh), lambda h: (0, h)),
            pl.BlockSpec((S, Dh), lambda h: (0, h)),
            pl.BlockSpec((Dr, S), lambda h: (h, 0)),
            pl.BlockSpec((S, Dr), lambda h: (0, 0)),
        ],
        out_specs=pl.BlockSpec((S, Dh), lambda h: (0, h)),
        compiler_params=pltpu.CompilerParams(
            dimension_semantics=("arbitrary",)),
    )(q, k, v, qrt, kr)


def _oproj_body(o_ref, wo_ref, out_ref):
    out_ref[0] = jnp.dot(o_ref[...], wo_ref[...],
                         preferred_element_type=jnp.float32)


def _oproj(o, Wo):
    return pl.pallas_call(
        _oproj_body,
        out_shape=jax.ShapeDtypeStruct((1, S, D), jnp.float32),
        in_specs=[pl.BlockSpec(memory_space=pltpu.VMEM)] * 2,
        out_specs=pl.BlockSpec(memory_space=pltpu.VMEM),
    )(o, Wo)


def kernel(x, Wdkv, Wuk, Wuv, Wq, Wqr, Wkr, Wo):
    k, v, qrt, kr = _proj_exchange(x, Wdkv, Wuk, Wuv, Wqr, Wkr)
    q = _qproj(x, Wq)
    o = _attn(q, k, v, qrt, kr)
    return _oproj(o, Wo)
